# baseline (speedup 1.0000x reference)
import functools

import jax
import jax.numpy as jnp
from jax import lax
from jax.experimental import pallas as pl
from jax.experimental.pallas import tpu as pltpu
from jax.experimental.pallas import tpu_sc as plsc

N = 16384
DIM = 64
OUT_D = DIM + 2

_info = plsc.get_sparse_core_info()
NC = _info.num_cores
NS = _info.num_subcores
L = _info.num_lanes
NW = NC * NS
B_PER_W = N // NW
CHUNK = 128
N_CHUNKS = B_PER_W // CHUNK


def _emb_kernel(vocab_hbm, sel_hbm, st_hbm, table_hbm,
                out_hbm, cola_hbm, colb_hbm,
                idx_v, sel_v, st_v, out_v, cola_v, colb_v, sem):
    wid = lax.axis_index("s") * NC + lax.axis_index("c")
    base = wid * B_PER_W

    pltpu.sync_copy(vocab_hbm.at[pl.ds(base, B_PER_W)], idx_v)
    pltpu.sync_copy(sel_hbm.at[pl.ds(base, B_PER_W)], sel_v)
    pltpu.sync_copy(st_hbm, st_v)

    copies = []
    for c in range(N_CHUNKS):
        copies.append(
            pltpu.async_copy(
                table_hbm.at[idx_v.at[pl.ds(c * CHUNK, CHUNK)]],
                out_v.at[pl.ds(c * CHUNK, CHUNK), :],
                sem,
            )
        )

    st00 = st_v[pl.ds(0 * L, L)]
    st01 = st_v[pl.ds(1 * L, L)]
    st10 = st_v[pl.ds(2 * L, L)]
    st11 = st_v[pl.ds(3 * L, L)]

    def body(i, carry):
        s = sel_v[pl.ds(i * L, L)]
        is0 = s == 0
        cola_v[pl.ds(i * L, L)] = jnp.where(is0, st00, st10)
        colb_v[pl.ds(i * L, L)] = jnp.where(is0, st01, st11)
        return carry

    lax.fori_loop(0, B_PER_W // L, body, 0)

    pltpu.sync_copy(cola_v, cola_hbm.at[pl.ds(base, B_PER_W)])
    pltpu.sync_copy(colb_v, colb_hbm.at[pl.ds(base, B_PER_W)])

    for cp in copies:
        cp.wait()

    pltpu.sync_copy(out_v, out_hbm.at[pl.ds(base, B_PER_W)])


@jax.jit
def _emb(vocab_ids, selector_ids, table, selector_table):
    mesh = plsc.VectorSubcoreMesh(core_axis_name="c", subcore_axis_name="s")
    st64 = jnp.repeat(selector_table.reshape(-1), L)
    table2 = table.reshape(table.shape[0] // 2, 2 * DIM)
    idx2 = vocab_ids >> 1
    par = vocab_ids & 1
    f = functools.partial(
        pl.kernel,
        mesh=mesh,
        out_type=(
            jax.ShapeDtypeStruct((N, 2 * DIM), jnp.float32),
            jax.ShapeDtypeStruct((N,), jnp.float32),
            jax.ShapeDtypeStruct((N,), jnp.float32),
        ),
        scratch_types=[
            pltpu.VMEM((B_PER_W,), jnp.int32),
            pltpu.VMEM((B_PER_W,), jnp.int32),
            pltpu.VMEM((4 * L,), jnp.float32),
            pltpu.VMEM((B_PER_W, 2 * DIM), jnp.float32),
            pltpu.VMEM((B_PER_W,), jnp.float32),
            pltpu.VMEM((B_PER_W,), jnp.float32),
            pltpu.SemaphoreType.DMA,
        ],
    )(_emb_kernel)
    pairs, cola, colb = f(idx2, selector_ids, st64, table2)
    emb = jnp.where((par == 1)[:, None], pairs[:, DIM:], pairs[:, :DIM])
    return jnp.concatenate((emb, cola[:, None], colb[:, None]), axis=1)


def kernel(vocab_ids, selector_ids, table, selector_table):
    return _emb(vocab_ids.astype(jnp.int32), selector_ids.astype(jnp.int32),
                table, selector_table)


# speedup vs baseline: 3.9343x; 3.9343x over previous
import functools

import jax
import jax.numpy as jnp
from jax import lax
from jax.experimental import pallas as pl
from jax.experimental.pallas import tpu as pltpu
from jax.experimental.pallas import tpu_sc as plsc

N = 16384
DIM = 64
OUT_D = DIM + 2
V = 1000000

_info = plsc.get_sparse_core_info()
NC = _info.num_cores
NS = _info.num_subcores
L = _info.num_lanes
NW = NC * NS

CW = 10752
NFULL = V // CW
TAIL = V - NFULL * CW
NCH = NFULL + 1
SELCW = 2048


def _emb_kernel(sv_hbm, perm_hbm, starts_hbm, selbits_hbm, st_hbm,
                tail_hbm, tableT_hbm, outT_hbm,
                sv_v, perm_v, starts_v, st_v, tail_v,
                slab0, slab1, slab2, stage_v, sem0, sem1, sem2):
    wid = lax.axis_index("s") * NC + lax.axis_index("c")
    d0 = 2 * wid

    pltpu.sync_copy(sv_hbm, sv_v)
    pltpu.sync_copy(perm_hbm, perm_v)
    pltpu.sync_copy(starts_hbm, starts_v)
    pltpu.sync_copy(st_hbm, st_v)
    pltpu.sync_copy(tail_hbm.at[pl.ds(d0, 2), :], tail_v)

    k16 = lax.iota(jnp.int32, L)
    zeros = k16 * 0
    ones = zeros + 1

    def extract(c):
        win = starts_v[pl.ds(16 * (c // 16), L)]
        return jnp.sum(jnp.where(k16 == c % 16, win, 0))

    def process(c, jlo, jhi, slab, width):
        g0 = jlo >> 4
        g1 = (jhi + 15) >> 4

        def g_body(g, carry):
            j = g * L + k16
            mask = (j >= jlo) & (j < jhi)
            vvec = sv_v[pl.ds(g * L, L)]
            pvec = perm_v[pl.ds(g * L, L)]
            vloc = jnp.clip(vvec - c * CW, 0, width - 1)
            a = plsc.load_gather(slab, [zeros, vloc])
            b = plsc.load_gather(slab, [ones, vloc])
            plsc.store_scatter(stage_v, [zeros, pvec], a, mask=mask)
            plsc.store_scatter(stage_v, [ones, pvec], b, mask=mask)
            return carry

        lax.fori_loop(g0, g1, g_body, 0)

    slabs = [slab0, slab1, slab2]
    sems = [sem0, sem1, sem2]

    def start(c, b):
        pltpu.async_copy(
            tableT_hbm.at[pl.ds(d0, 2), pl.ds(c * CW, CW)],
            slabs[b], sems[b])

    def drain(b):
        pltpu.make_async_copy(
            tableT_hbm.at[pl.ds(0, 2), pl.ds(0, CW)],
            slabs[b], sems[b]).wait()

    start(0, 0)
    start(1, 1)

    def triple_body(g, jlo):
        c0 = 3 * g

        def third(b, jlo_b):
            c = c0 + b
            drain(b)

            @pl.when(c + 2 < NFULL)
            def _():
                start(c + 2, (b + 2) % 3)

            jhi = extract(c + 1)
            process(c, jlo_b, jhi, slabs[b], CW)
            return jhi

        jlo = third(0, jlo)
        jlo = third(1, jlo)
        jlo = third(2, jlo)
        return jlo

    jlo = lax.fori_loop(0, NFULL // 3, triple_body, 0)
    process(NFULL, jlo, extract(NCH), tail_v, TAIL)

    stx0 = st_v[pl.ds(0 * L, L)]
    stx1 = st_v[pl.ds(1 * L, L)]
    sty0 = st_v[pl.ds(2 * L, L)]
    sty1 = st_v[pl.ds(3 * L, L)]

    @pl.when(wid < 2)
    def _():
        is1 = wid == 1
        hi = jnp.where(is1, stx1, stx0)
        lo = jnp.where(is1, sty1, sty0)

        def sel_chunk(ch, carry):
            pltpu.sync_copy(selbits_hbm.at[pl.ds(ch * SELCW, SELCW)],
                            slab0.at[0, pl.ds(0, SELCW)])

            def sel_body(i, carry2):
                s = plsc.bitcast(slab0[0, pl.ds(i * L, L)], jnp.int32)
                slab0[1, pl.ds(i * L, L)] = jnp.where(s == 0, hi, lo)
                return carry2

            lax.fori_loop(0, SELCW // L, sel_body, 0)
            pltpu.sync_copy(
                slab0.at[pl.ds(1, 1), pl.ds(0, SELCW)],
                outT_hbm.at[pl.ds(DIM + wid, 1), pl.ds(ch * SELCW, SELCW)])
            return carry

        lax.fori_loop(0, N // SELCW, sel_chunk, 0)

    pltpu.sync_copy(stage_v, outT_hbm.at[pl.ds(d0, 2), :])


@jax.jit
def _emb(vocab_ids, selector_ids, table, selector_table):
    mesh = plsc.VectorSubcoreMesh(core_axis_name="c", subcore_axis_name="s")
    tableT = table.T
    tail64 = table[NFULL * CW:, :].T
    st64 = jnp.repeat(selector_table.reshape(-1), L)
    selbits = lax.bitcast_convert_type(
        selector_ids.astype(jnp.int32), jnp.float32)
    sv, order = lax.sort_key_val(vocab_ids.astype(jnp.int32),
                                 jnp.arange(N, dtype=jnp.int32))
    bounds = jnp.concatenate([
        jnp.arange(NCH, dtype=jnp.int32) * CW,
        jnp.array([V], dtype=jnp.int32),
    ])
    starts = jnp.searchsorted(sv, bounds).astype(jnp.int32)
    starts = jnp.pad(starts, (0, 96 - (NCH + 1)))
    f = functools.partial(
        pl.kernel,
        mesh=mesh,
        out_type=jax.ShapeDtypeStruct((OUT_D, N), jnp.float32),
        scratch_types=[
            pltpu.VMEM((N,), jnp.int32),
            pltpu.VMEM((N,), jnp.int32),
            pltpu.VMEM((96,), jnp.int32),
            pltpu.VMEM((4 * L,), jnp.float32),
            pltpu.VMEM((2, TAIL), jnp.float32),
            pltpu.VMEM((2, CW), jnp.float32),
            pltpu.VMEM((2, CW), jnp.float32),
            pltpu.VMEM((2, CW), jnp.float32),
            pltpu.VMEM((2, N), jnp.float32),
            pltpu.SemaphoreType.DMA,
            pltpu.SemaphoreType.DMA,
            pltpu.SemaphoreType.DMA,
        ],
        compiler_params=pltpu.CompilerParams(needs_layout_passes=False),
    )(_emb_kernel)
    outT = f(sv, order, starts, selbits, st64, tail64, tableT)
    return outT.T


def kernel(vocab_ids, selector_ids, table, selector_table):
    return _emb(vocab_ids.astype(jnp.int32), selector_ids.astype(jnp.int32),
                table, selector_table)
